# baseline (speedup 1.0000x reference)
import functools

import jax
import jax.numpy as jnp
from jax import lax
from jax.experimental import pallas as pl
from jax.experimental.pallas import tpu as pltpu
from jax.experimental.pallas import tpu_sc as plsc

VOCAB = 100000
HIDDEN = 128
BATCH = 1024
SEQ = 200
N_IDX = BATCH * SEQ
NW = 32
PER_W = N_IDX // NW
CHUNK = 100
N_CHUNK = PER_W // CHUNK
K = 4
GRP = K * CHUNK
N_GRP = N_CHUNK // K


def _make_sc_gather():
    mesh = plsc.VectorSubcoreMesh(core_axis_name="c", subcore_axis_name="s")

    @functools.partial(
        pl.kernel,
        mesh=mesh,
        out_type=jax.ShapeDtypeStruct((N_IDX, HIDDEN), jnp.float32),
        scratch_types=[
            pltpu.VMEM((N_CHUNK, CHUNK), jnp.int32),
            pltpu.VMEM((2, GRP, HIDDEN), jnp.float32),
            pltpu.SemaphoreType.DMA,
            pltpu.SemaphoreType.DMA,
            pltpu.SemaphoreType.DMA,
            pltpu.SemaphoreType.DMA,
        ],
    )
    def gather_kernel(idx_hbm, table_hbm, out_hbm, idx_v, rows_v, sg0, sg1, so0, so1):
        sg = (sg0, sg1)
        so = (so0, so1)
        wid = lax.axis_index("s") * 2 + lax.axis_index("c")
        base = wid * PER_W
        pltpu.sync_copy(idx_hbm.at[wid], idx_v)

        def issue_gather(i, b, p):
            pltpu.async_copy(
                table_hbm.at[idx_v.at[i]],
                rows_v.at[p].at[pl.ds(b * CHUNK, CHUNK)],
                sg[p],
            )

        def wait_gather(i, b, p):
            pltpu.make_async_copy(
                table_hbm.at[idx_v.at[i]],
                rows_v.at[p].at[pl.ds(b * CHUNK, CHUNK)],
                sg[p],
            ).wait()

        def issue_out(j, p):
            pltpu.async_copy(
                rows_v.at[p], out_hbm.at[pl.ds(base + j * GRP, GRP)], so[p]
            )

        def wait_out(j, p):
            pltpu.make_async_copy(
                rows_v.at[p], out_hbm.at[pl.ds(base + j * GRP, GRP)], so[p]
            ).wait()

        for b in range(K):
            issue_gather(b, b, 0)

        def body(j2, carry):
            for p in range(2):
                j = j2 * 2 + p
                for b in range(K):
                    wait_gather(j * K + b, b, p)
                issue_out(j, p)
                @pl.when(j < N_GRP - 1)
                def _():
                    q = 1 - p
                    @pl.when(j > 0)
                    def _():
                        wait_out(j - 1, q)
                    for b in range(K):
                        issue_gather((j + 1) * K + b, b, q)
            return carry

        lax.fori_loop(0, N_GRP // 2, body, 0)

        wait_out(N_GRP - 2, 0)
        wait_out(N_GRP - 1, 1)

    return gather_kernel


_sc_gather = _make_sc_gather()


def kernel(x, word_embeddings_weight):
    idx = x.reshape(NW, N_CHUNK, CHUNK).astype(jnp.int32)
    gathered = _sc_gather(idx, word_embeddings_weight)
    return jnp.transpose(gathered.reshape(BATCH, SEQ, HIDDEN), (0, 2, 1))


# speedup vs baseline: 1.0378x; 1.0378x over previous
import functools

import jax
import jax.numpy as jnp
from jax import lax
from jax.experimental import pallas as pl
from jax.experimental.pallas import tpu as pltpu
from jax.experimental.pallas import tpu_sc as plsc

VOCAB = 100000
HIDDEN = 128
BATCH = 1024
SEQ = 200
N_IDX = BATCH * SEQ
NW = 32
PER_W = N_IDX // NW
CHUNK = 80
N_CHUNK = PER_W // CHUNK
K = 5
N_GRP = N_CHUNK // K


def _make_sc_gather():
    mesh = plsc.VectorSubcoreMesh(core_axis_name="c", subcore_axis_name="s")

    @functools.partial(
        pl.kernel,
        mesh=mesh,
        out_type=jax.ShapeDtypeStruct((N_IDX, HIDDEN), jnp.float32),
        scratch_types=[
            pltpu.VMEM((N_CHUNK, CHUNK), jnp.int32),
            pltpu.VMEM((2 * K, CHUNK, HIDDEN), jnp.float32),
            pltpu.SemaphoreType.DMA,
            pltpu.SemaphoreType.DMA,
            pltpu.SemaphoreType.DMA,
            pltpu.SemaphoreType.DMA,
        ],
    )
    def gather_kernel(idx_hbm, table_hbm, out_hbm, idx_v, rows_v, sg0, sg1, so0, so1):
        sg = (sg0, sg1)
        so = (so0, so1)
        wid = lax.axis_index("s") * 2 + lax.axis_index("c")
        base = wid * PER_W
        pltpu.sync_copy(idx_hbm.at[wid], idx_v)

        def issue_gather(i, buf, p):
            pltpu.async_copy(table_hbm.at[idx_v.at[i]], rows_v.at[buf], sg[p])

        def wait_gather(i, buf, p):
            pltpu.make_async_copy(
                table_hbm.at[idx_v.at[i]], rows_v.at[buf], sg[p]
            ).wait()

        def issue_out(i, buf, p):
            pltpu.async_copy(
                rows_v.at[buf], out_hbm.at[pl.ds(base + i * CHUNK, CHUNK)], so[p]
            )

        def wait_out(i, buf, p):
            pltpu.make_async_copy(
                rows_v.at[buf], out_hbm.at[pl.ds(base + i * CHUNK, CHUNK)], so[p]
            ).wait()

        for b in range(K):
            issue_gather(b, b, 0)

        def body(j2, carry):
            for p in range(2):
                j = j2 * 2 + p
                q = 1 - p
                for b in range(K):
                    i = j * K + b
                    wait_gather(i, p * K + b, p)
                    issue_out(i, p * K + b, p)
                    @pl.when(j < N_GRP - 1)
                    def _():
                        @pl.when(j > 0)
                        def _():
                            wait_out((j - 1) * K + b, q * K + b, q)
                        issue_gather((j + 1) * K + b, q * K + b, q)
            return carry

        lax.fori_loop(0, N_GRP // 2, body, 0)

        for p in range(2):
            j = N_GRP - 2 + p
            for b in range(K):
                wait_out(j * K + b, p * K + b, p)

    return gather_kernel


_sc_gather = _make_sc_gather()


def kernel(x, word_embeddings_weight):
    idx = x.reshape(NW, N_CHUNK, CHUNK).astype(jnp.int32)
    gathered = _sc_gather(idx, word_embeddings_weight)
    return jnp.transpose(gathered.reshape(BATCH, SEQ, HIDDEN), (0, 2, 1))
